# baseline (speedup 1.0000x reference)
import functools

import jax
import jax.numpy as jnp
from jax import lax
from jax.experimental import pallas as pl
from jax.experimental.pallas import tpu as pltpu
from jax.experimental.pallas import tpu_sc as plsc

_B = 16384
_K = 16
_NC = 2
_NS = 16
_NW = _NC * _NS
_BPW = _B // _NW
_CHUNK = 128
_NCHUNK = _BPW // _CHUNK


def _mf_body(uidx_hbm, iidx_hbm, w_hbm, h_hbm, out_hbm,
             uidx_v, iidx_v, urows_v, irows_v, out_v, sem):
    wid = lax.axis_index("s") * _NC + lax.axis_index("c")
    base = wid * _BPW

    for j in range(_NCHUNK):
        pltpu.sync_copy(uidx_hbm.at[pl.ds(base + j * _CHUNK, _CHUNK)],
                        uidx_v.at[j])
        pltpu.sync_copy(iidx_hbm.at[pl.ds(base + j * _CHUNK, _CHUNK)],
                        iidx_v.at[j])

    copies = []
    for j in range(_NCHUNK):
        copies.append(pltpu.async_copy(
            w_hbm.at[uidx_v.at[j]],
            urows_v.at[pl.ds(j * _CHUNK, _CHUNK), :], sem))
        copies.append(pltpu.async_copy(
            h_hbm.at[iidx_v.at[j]],
            irows_v.at[pl.ds(j * _CHUNK, _CHUNK), :], sem))
    for c in copies:
        c.wait()

    lane = lax.iota(jnp.int32, 16)

    def block(b, carry):
        rows = b * 16 + lane
        acc = jnp.zeros((16,), jnp.float32)
        for k in range(_K):
            col = jnp.full((16,), k, jnp.int32)
            u_k = plsc.load_gather(urows_v, [rows, col])
            v_k = plsc.load_gather(irows_v, [rows, col])
            acc = acc + u_k * v_k
        out_v[pl.ds(b * 16, 16)] = 1.0 / (1.0 + jnp.exp(-acc))
        return carry

    lax.fori_loop(0, _BPW // 16, block, 0)

    pltpu.sync_copy(out_v, out_hbm.at[pl.ds(base, _BPW)])


@functools.partial(jax.jit, donate_argnums=())
def _mf_forward(uidx, iidx, w, h):
    mesh = plsc.VectorSubcoreMesh(core_axis_name="c", subcore_axis_name="s",
                                  num_cores=_NC, num_subcores=_NS)
    run = pl.kernel(
        _mf_body,
        out_type=jax.ShapeDtypeStruct((_B,), jnp.float32),
        mesh=mesh,
        compiler_params=pltpu.CompilerParams(needs_layout_passes=False,
                                             use_tc_tiling_on_sc=False),
        scratch_types=[
            pltpu.VMEM((_NCHUNK, _CHUNK), jnp.int32),
            pltpu.VMEM((_NCHUNK, _CHUNK), jnp.int32),
            pltpu.VMEM((_BPW, _K), jnp.float32),
            pltpu.VMEM((_BPW, _K), jnp.float32),
            pltpu.VMEM((_BPW,), jnp.float32),
            pltpu.SemaphoreType.DMA,
        ],
    )
    return run(uidx, iidx, w, h)


def kernel(x, W, H):
    uidx = x[:, 0]
    iidx = x[:, 1]
    return _mf_forward(uidx, iidx, W, H)
